# baseline (speedup 1.0000x reference)
import functools

import numpy as np
import jax
import jax.numpy as jnp
from jax import lax
from jax.experimental import pallas as pl
from jax.experimental.pallas import tpu as pltpu
from jax.experimental.pallas import tpu_sc as plsc

_NUM_NODES = 10000
_INITIAL_SIZE = 256
_KEEP = 0.8
_N = _NUM_NODES * _INITIAL_SIZE
_NW = 32
_PER_W = _N // _NW
_CH = 16000
_NCHUNK = _PER_W // _CH


def _threefry2x32(k1, k2, x0, x1):
    def rotl(x, r):
        return ((x << np.uint32(r)) | (x >> np.uint32(32 - r))).astype(np.uint32)
    ks0, ks1 = np.uint32(k1), np.uint32(k2)
    ks2 = np.uint32(ks0 ^ ks1 ^ np.uint32(0x1BD11BDA))
    ks = [ks0, ks1, ks2]
    x0 = (x0 + ks0).astype(np.uint32)
    x1 = (x1 + ks1).astype(np.uint32)
    rounds = [[13, 15, 26, 6], [17, 29, 16, 24]]
    for i in range(5):
        for r in rounds[i % 2]:
            x0 = (x0 + x1).astype(np.uint32)
            x1 = rotl(x1, r)
            x1 = (x1 ^ x0).astype(np.uint32)
        x0 = (x0 + ks[(i + 1) % 3]).astype(np.uint32)
        x1 = (x1 + ks[(i + 2) % 3] + np.uint32(i + 1)).astype(np.uint32)
    return x0, x1


def _bernoulli_mask(seed, p, n):
    k1 = np.uint32(np.int64(seed) >> np.int64(32))
    k2 = np.uint32(np.int64(seed) & np.int64(0xFFFFFFFF))
    lo = np.arange(n, dtype=np.uint32)
    hi = np.zeros(n, dtype=np.uint32)
    o0, o1 = _threefry2x32(k1, k2, hi, lo)
    bits = o0 ^ o1
    float_bits = ((bits >> np.uint32(9)) | np.uint32(0x3F800000)).astype(np.uint32)
    u = np.maximum(np.float32(0.0), float_bits.view(np.float32) - np.float32(1.0))
    return u < np.float32(p)


_MASK_SCALED = np.where(_bernoulli_mask(42, _KEEP, _N),
                        np.float32(1.0 / _KEEP), np.float32(0.0))

_mesh = plsc.VectorSubcoreMesh(core_axis_name="c", subcore_axis_name="s")


@functools.partial(
    pl.kernel, mesh=_mesh,
    out_type=jax.ShapeDtypeStruct((_N,), jnp.float32),
    scratch_types=[
        pltpu.VMEM((_CH,), jnp.float32),
        pltpu.VMEM((_CH,), jnp.float32),
    ],
)
def _sc_dropout(emb_hbm, mask_hbm, out_hbm, emb_v, mask_v):
    wid = lax.axis_index("s") * 2 + lax.axis_index("c")
    base = wid * _PER_W
    for c in range(_NCHUNK):
        off = base + c * _CH
        pltpu.sync_copy(emb_hbm.at[pl.ds(off, _CH)], emb_v)
        pltpu.sync_copy(mask_hbm.at[pl.ds(off, _CH)], mask_v)

        def body(j, carry):
            s = j * 16
            emb_v[pl.ds(s, 16)] = emb_v[pl.ds(s, 16)] * mask_v[pl.ds(s, 16)]
            return carry

        lax.fori_loop(0, _CH // 16, body, 0)
        pltpu.sync_copy(emb_v, out_hbm.at[pl.ds(off, _CH)])


def kernel(adj_t, emb):
    del adj_t
    out = _sc_dropout(emb.reshape(_N), jnp.asarray(_MASK_SCALED))
    return out.reshape(_NUM_NODES, _INITIAL_SIZE)


# speedup vs baseline: 10.1883x; 10.1883x over previous
import numpy as np
import jax
import jax.numpy as jnp
from jax.experimental import pallas as pl
from jax.experimental.pallas import tpu as pltpu

_NUM_NODES = 10000
_INITIAL_SIZE = 256
_KEEP = 0.8

_NCH = 10
_R = _NUM_NODES // _NCH
_K = 4


def _threefry2x32(k1, k2, x0, x1):
    def rotl(x, r):
        return ((x << np.uint32(r)) | (x >> np.uint32(32 - r))).astype(np.uint32)
    ks0, ks1 = np.uint32(k1), np.uint32(k2)
    ks2 = np.uint32(ks0 ^ ks1 ^ np.uint32(0x1BD11BDA))
    ks = [ks0, ks1, ks2]
    x0 = (x0 + ks0).astype(np.uint32)
    x1 = (x1 + ks1).astype(np.uint32)
    rounds = [[13, 15, 26, 6], [17, 29, 16, 24]]
    for i in range(5):
        for r in rounds[i % 2]:
            x0 = (x0 + x1).astype(np.uint32)
            x1 = rotl(x1, r)
            x1 = (x1 ^ x0).astype(np.uint32)
        x0 = (x0 + ks[(i + 1) % 3]).astype(np.uint32)
        x1 = (x1 + ks[(i + 2) % 3] + np.uint32(i + 1)).astype(np.uint32)
    return x0, x1


def _bernoulli_mask(seed, p, shape):
    n = int(np.prod(shape))
    k1 = np.uint32(np.int64(seed) >> np.int64(32))
    k2 = np.uint32(np.int64(seed) & np.int64(0xFFFFFFFF))
    lo = np.arange(n, dtype=np.uint32)
    hi = np.zeros(n, dtype=np.uint32)
    o0, o1 = _threefry2x32(k1, k2, hi, lo)
    bits = o0 ^ o1
    float_bits = ((bits >> np.uint32(9)) | np.uint32(0x3F800000)).astype(np.uint32)
    u = np.maximum(np.float32(0.0), float_bits.view(np.float32) - np.float32(1.0))
    return (u < np.float32(p)).reshape(shape)


_MASK_I8 = _bernoulli_mask(42, _KEEP, (_NUM_NODES, _INITIAL_SIZE)).astype(np.int8)


def _body(emb_hbm, mask_hbm, out_hbm, ebuf, mbuf, obuf, esem, msem, osem):
    def in_copies(c):
        slot = c % _K
        e = pltpu.make_async_copy(
            emb_hbm.at[pl.ds(c * _R, _R)], ebuf.at[slot], esem.at[slot])
        m = pltpu.make_async_copy(
            mask_hbm.at[pl.ds(c * _R, _R)], mbuf.at[slot], msem.at[slot])
        return e, m

    def out_copy(c):
        slot = c % _K
        return pltpu.make_async_copy(
            obuf.at[slot], out_hbm.at[pl.ds(c * _R, _R)], osem.at[slot])

    for c in range(_K):
        e, m = in_copies(c)
        e.start()
        m.start()
    for c in range(_NCH):
        slot = c % _K
        e, m = in_copies(c)
        e.wait()
        m.wait()
        if c >= _K:
            out_copy(c - _K).wait()
        obuf[slot] = jnp.where(
            mbuf[slot] != 0, ebuf[slot] * (1.0 / _KEEP), 0.0)
        out_copy(c).start()
        if c + _K < _NCH:
            e2, m2 = in_copies(c + _K)
            e2.start()
            m2.start()
    for c in range(_NCH - _K, _NCH):
        out_copy(c).wait()


def kernel(adj_t, emb):
    del adj_t
    return pl.pallas_call(
        _body,
        in_specs=[
            pl.BlockSpec(memory_space=pl.ANY),
            pl.BlockSpec(memory_space=pl.ANY),
        ],
        out_specs=pl.BlockSpec(memory_space=pl.ANY),
        out_shape=jax.ShapeDtypeStruct((_NUM_NODES, _INITIAL_SIZE),
                                       jnp.float32),
        scratch_shapes=[
            pltpu.VMEM((_K, _R, _INITIAL_SIZE), jnp.float32),
            pltpu.VMEM((_K, _R, _INITIAL_SIZE), jnp.int8),
            pltpu.VMEM((_K, _R, _INITIAL_SIZE), jnp.float32),
            pltpu.SemaphoreType.DMA((_K,)),
            pltpu.SemaphoreType.DMA((_K,)),
            pltpu.SemaphoreType.DMA((_K,)),
        ],
    )(emb, _MASK_I8)


# speedup vs baseline: 10.3012x; 1.0111x over previous
import numpy as np
import jax
import jax.numpy as jnp
from jax.experimental import pallas as pl
from jax.experimental.pallas import tpu as pltpu

_NUM_NODES = 10000
_INITIAL_SIZE = 256
_KEEP = 0.8

_NCH = 5
_R = _NUM_NODES // _NCH
_K = 3


def _threefry2x32(k1, k2, x0, x1):
    def rotl(x, r):
        return ((x << np.uint32(r)) | (x >> np.uint32(32 - r))).astype(np.uint32)
    ks0, ks1 = np.uint32(k1), np.uint32(k2)
    ks2 = np.uint32(ks0 ^ ks1 ^ np.uint32(0x1BD11BDA))
    ks = [ks0, ks1, ks2]
    x0 = (x0 + ks0).astype(np.uint32)
    x1 = (x1 + ks1).astype(np.uint32)
    rounds = [[13, 15, 26, 6], [17, 29, 16, 24]]
    for i in range(5):
        for r in rounds[i % 2]:
            x0 = (x0 + x1).astype(np.uint32)
            x1 = rotl(x1, r)
            x1 = (x1 ^ x0).astype(np.uint32)
        x0 = (x0 + ks[(i + 1) % 3]).astype(np.uint32)
        x1 = (x1 + ks[(i + 2) % 3] + np.uint32(i + 1)).astype(np.uint32)
    return x0, x1


def _bernoulli_mask(seed, p, shape):
    n = int(np.prod(shape))
    k1 = np.uint32(np.int64(seed) >> np.int64(32))
    k2 = np.uint32(np.int64(seed) & np.int64(0xFFFFFFFF))
    lo = np.arange(n, dtype=np.uint32)
    hi = np.zeros(n, dtype=np.uint32)
    o0, o1 = _threefry2x32(k1, k2, hi, lo)
    bits = o0 ^ o1
    float_bits = ((bits >> np.uint32(9)) | np.uint32(0x3F800000)).astype(np.uint32)
    u = np.maximum(np.float32(0.0), float_bits.view(np.float32) - np.float32(1.0))
    return (u < np.float32(p)).reshape(shape)


_MASK_I8 = _bernoulli_mask(42, _KEEP, (_NUM_NODES, _INITIAL_SIZE)).astype(np.int8)


def _body(emb_hbm, mask_hbm, out_hbm, ebuf, mbuf, obuf, esem, msem, osem):
    def in_copies(c):
        slot = c % _K
        e = pltpu.make_async_copy(
            emb_hbm.at[pl.ds(c * _R, _R)], ebuf.at[slot], esem.at[slot])
        m = pltpu.make_async_copy(
            mask_hbm.at[pl.ds(c * _R, _R)], mbuf.at[slot], msem.at[slot])
        return e, m

    def out_copy(c):
        slot = c % _K
        return pltpu.make_async_copy(
            obuf.at[slot], out_hbm.at[pl.ds(c * _R, _R)], osem.at[slot])

    for c in range(_K):
        e, m = in_copies(c)
        e.start()
        m.start()
    for c in range(_NCH):
        slot = c % _K
        e, m = in_copies(c)
        e.wait()
        m.wait()
        if c >= _K:
            out_copy(c - _K).wait()
        obuf[slot] = jnp.where(
            mbuf[slot] != 0, ebuf[slot] * (1.0 / _KEEP), 0.0)
        out_copy(c).start()
        if c + _K < _NCH:
            e2, m2 = in_copies(c + _K)
            e2.start()
            m2.start()
    for c in range(_NCH - _K, _NCH):
        out_copy(c).wait()


def kernel(adj_t, emb):
    del adj_t
    return pl.pallas_call(
        _body,
        in_specs=[
            pl.BlockSpec(memory_space=pl.ANY),
            pl.BlockSpec(memory_space=pl.ANY),
        ],
        out_specs=pl.BlockSpec(memory_space=pl.ANY),
        out_shape=jax.ShapeDtypeStruct((_NUM_NODES, _INITIAL_SIZE),
                                       jnp.float32),
        scratch_shapes=[
            pltpu.VMEM((_K, _R, _INITIAL_SIZE), jnp.float32),
            pltpu.VMEM((_K, _R, _INITIAL_SIZE), jnp.int8),
            pltpu.VMEM((_K, _R, _INITIAL_SIZE), jnp.float32),
            pltpu.SemaphoreType.DMA((_K,)),
            pltpu.SemaphoreType.DMA((_K,)),
            pltpu.SemaphoreType.DMA((_K,)),
        ],
    )(emb, _MASK_I8)


# speedup vs baseline: 11.5810x; 1.1242x over previous
import numpy as np
import jax
import jax.numpy as jnp
from jax.experimental import pallas as pl
from jax.experimental.pallas import tpu as pltpu

_NUM_NODES = 10000
_INITIAL_SIZE = 256
_KEEP = 0.8

_NCH = 2
_R = _NUM_NODES // _NCH
_K = 2


def _threefry2x32(k1, k2, x0, x1):
    def rotl(x, r):
        return ((x << np.uint32(r)) | (x >> np.uint32(32 - r))).astype(np.uint32)
    ks0, ks1 = np.uint32(k1), np.uint32(k2)
    ks2 = np.uint32(ks0 ^ ks1 ^ np.uint32(0x1BD11BDA))
    ks = [ks0, ks1, ks2]
    x0 = (x0 + ks0).astype(np.uint32)
    x1 = (x1 + ks1).astype(np.uint32)
    rounds = [[13, 15, 26, 6], [17, 29, 16, 24]]
    for i in range(5):
        for r in rounds[i % 2]:
            x0 = (x0 + x1).astype(np.uint32)
            x1 = rotl(x1, r)
            x1 = (x1 ^ x0).astype(np.uint32)
        x0 = (x0 + ks[(i + 1) % 3]).astype(np.uint32)
        x1 = (x1 + ks[(i + 2) % 3] + np.uint32(i + 1)).astype(np.uint32)
    return x0, x1


def _bernoulli_mask(seed, p, shape):
    n = int(np.prod(shape))
    k1 = np.uint32(np.int64(seed) >> np.int64(32))
    k2 = np.uint32(np.int64(seed) & np.int64(0xFFFFFFFF))
    lo = np.arange(n, dtype=np.uint32)
    hi = np.zeros(n, dtype=np.uint32)
    o0, o1 = _threefry2x32(k1, k2, hi, lo)
    bits = o0 ^ o1
    float_bits = ((bits >> np.uint32(9)) | np.uint32(0x3F800000)).astype(np.uint32)
    u = np.maximum(np.float32(0.0), float_bits.view(np.float32) - np.float32(1.0))
    return (u < np.float32(p)).reshape(shape)


_MASK_I8 = _bernoulli_mask(42, _KEEP, (_NUM_NODES, _INITIAL_SIZE)).astype(np.int8)


def _body(emb_hbm, mask_hbm, out_hbm, ebuf, mbuf, obuf, esem, msem, osem):
    def in_copies(c):
        slot = c % _K
        e = pltpu.make_async_copy(
            emb_hbm.at[pl.ds(c * _R, _R)], ebuf.at[slot], esem.at[slot])
        m = pltpu.make_async_copy(
            mask_hbm.at[pl.ds(c * _R, _R)], mbuf.at[slot], msem.at[slot])
        return e, m

    def out_copy(c):
        slot = c % _K
        return pltpu.make_async_copy(
            obuf.at[slot], out_hbm.at[pl.ds(c * _R, _R)], osem.at[slot])

    for c in range(_K):
        e, m = in_copies(c)
        e.start()
        m.start()
    for c in range(_NCH):
        slot = c % _K
        e, m = in_copies(c)
        e.wait()
        m.wait()
        if c >= _K:
            out_copy(c - _K).wait()
        obuf[slot] = jnp.where(
            mbuf[slot] != 0, ebuf[slot] * (1.0 / _KEEP), 0.0)
        out_copy(c).start()
        if c + _K < _NCH:
            e2, m2 = in_copies(c + _K)
            e2.start()
            m2.start()
    for c in range(_NCH - _K, _NCH):
        out_copy(c).wait()


def kernel(adj_t, emb):
    del adj_t
    return pl.pallas_call(
        _body,
        in_specs=[
            pl.BlockSpec(memory_space=pl.ANY),
            pl.BlockSpec(memory_space=pl.ANY),
        ],
        out_specs=pl.BlockSpec(memory_space=pl.ANY),
        out_shape=jax.ShapeDtypeStruct((_NUM_NODES, _INITIAL_SIZE),
                                       jnp.float32),
        scratch_shapes=[
            pltpu.VMEM((_K, _R, _INITIAL_SIZE), jnp.float32),
            pltpu.VMEM((_K, _R, _INITIAL_SIZE), jnp.int8),
            pltpu.VMEM((_K, _R, _INITIAL_SIZE), jnp.float32),
            pltpu.SemaphoreType.DMA((_K,)),
            pltpu.SemaphoreType.DMA((_K,)),
            pltpu.SemaphoreType.DMA((_K,)),
        ],
    )(emb, _MASK_I8)
